# baseline (speedup 1.0000x reference)
import functools

import jax
import jax.numpy as jnp
from jax.experimental import pallas as pl

KS = 15
XP = 16
S0 = 0.6
GN_EPS = 1e-5
LRELU = 0.1


def _ceil_to(x, m):
    return (x + m - 1) // m * m


def _kpconv_body(infl_ref, nf_ref, w_ref, o_ref, *, K, C, B):
    parts = []
    for x in range(KS):
        def kstep(k, acc, x=x):
            return acc + infl_ref[k][:, x:x + 1] * nf_ref[k]
        parts.append(
            jax.lax.fori_loop(0, K, kstep, jnp.zeros((B, C), jnp.float32)))
    tmp = jnp.concatenate(parts, axis=1)
    o_ref[...] = jax.lax.dot_general(
        tmp, w_ref[...], dimension_numbers=(((1,), (0,)), ((), ())),
        preferred_element_type=jnp.float32)


def _kpconv_pallas(infl, nf, w_flat, B=512):
    K, Np, _ = infl.shape
    C = nf.shape[2]
    D = w_flat.shape[1]
    grid = Np // B
    return pl.pallas_call(
        functools.partial(_kpconv_body, K=K, C=C, B=B),
        grid=(grid,),
        in_specs=[
            pl.BlockSpec((K, B, XP), lambda i: (0, i, 0)),
            pl.BlockSpec((K, B, C), lambda i: (0, i, 0)),
            pl.BlockSpec((KS * C, D), lambda i: (0, 0)),
        ],
        out_specs=pl.BlockSpec((B, D), lambda i: (i, 0)),
        out_shape=jax.ShapeDtypeStruct((Np, D), jnp.float32),
    )(infl, nf, w_flat)


def _mm_body(x_ref, w_ref, sc_ref, sh_ref, o_ref, *, fuse_act):
    x = x_ref[...]
    if fuse_act:
        x = x * sc_ref[...] + sh_ref[...]
        x = jnp.where(x >= 0, x, LRELU * x)
    o_ref[...] = jax.lax.dot_general(
        x, w_ref[...], dimension_numbers=(((1,), (0,)), ((), ())),
        preferred_element_type=jnp.float32)


def _mm_pallas(x, w, scale=None, shift=None, B=1024):
    n_in = x.shape[0]
    Np = _ceil_to(n_in, B)
    if Np != n_in:
        x = jnp.pad(x, ((0, Np - n_in), (0, 0)))
    Cin = x.shape[1]
    D = w.shape[1]
    fuse = scale is not None
    if not fuse:
        scale = jnp.ones((1, Cin), jnp.float32)
        shift = jnp.zeros((1, Cin), jnp.float32)
    else:
        scale = scale.reshape(1, Cin)
        shift = shift.reshape(1, Cin)
    grid = Np // B
    return pl.pallas_call(
        functools.partial(_mm_body, fuse_act=fuse),
        grid=(grid,),
        in_specs=[
            pl.BlockSpec((B, Cin), lambda i: (i, 0)),
            pl.BlockSpec((Cin, D), lambda i: (0, 0)),
            pl.BlockSpec((1, Cin), lambda i: (0, 0)),
            pl.BlockSpec((1, Cin), lambda i: (0, 0)),
        ],
        out_specs=pl.BlockSpec((B, D), lambda i: (i, 0)),
        out_shape=jax.ShapeDtypeStruct((Np, D), jnp.float32),
    )(x, w, scale, shift)


def _gn_affine(x_valid, gamma, beta, groups=8):
    n, c = x_valid.shape
    gs = c // groups
    xg = x_valid.reshape(n, groups, gs)
    mean = xg.mean(axis=(0, 2))
    var = xg.var(axis=(0, 2))
    rs = jax.lax.rsqrt(var + GN_EPS)
    scale = jnp.repeat(rs, gs) * gamma
    shift = beta - jnp.repeat(mean * rs, gs) * gamma
    return scale, shift


def _lrelu_jnp(x):
    return jnp.where(x >= 0, x, LRELU * x)


def _influence(q_points, s_points, neighbors, kpts, sigma, Np):
    N, K = neighbors.shape
    nb = s_points[neighbors]
    diffs = nb - q_points[:, None, :]
    sq = ((diffs * diffs).sum(-1)[:, :, None]
          + (kpts * kpts).sum(-1)[None, None, :]
          - 2.0 * jnp.einsum('mki,xi->mkx', diffs, kpts))
    dist = jnp.sqrt(jnp.maximum(sq, 1e-12))
    infl = jnp.maximum(0.0, 1.0 - dist / sigma)
    infl = jnp.transpose(infl, (1, 0, 2))
    infl = jnp.pad(infl, ((0, 0), (0, Np - N), (0, XP - KS)))
    return infl


def _gather_kmajor(feats, neighbors, Np):
    N, K = neighbors.shape
    g = feats[neighbors.T]
    return jnp.pad(g, ((0, 0), (0, Np - N), (0, 0)))


def _kpconv_layer(feats, infl, neighbors, w, Np, N, in_scale=None,
                  in_shift=None):
    if in_scale is not None:
        feats = _lrelu_jnp(feats * in_scale + in_shift)
    nf = _gather_kmajor(feats, neighbors, Np)
    C = nf.shape[2]
    w_flat = w.reshape(KS * C, w.shape[2])
    out = _kpconv_pallas(infl, nf, w_flat)
    return out[:N]


def _pad_rows(x, Np):
    return jnp.pad(x, ((0, Np - x.shape[0]), (0, 0)))


def _res_block(p, s_feats, infl, neighbors, Np_q, Np_s, strided):
    N_q = neighbors.shape[0]
    N_s = s_feats.shape[0]
    x = _mm_pallas(_pad_rows(s_feats, Np_s), p['u1_W'])[:N_s]
    sc1, sh1 = _gn_affine(x, p['u1_g'], p['u1_b'])
    x_act = _lrelu_jnp(x * sc1 + sh1)
    kp = _kpconv_layer(x_act, infl, neighbors, p['kp_W'], Np_q, N_q)
    sc2, sh2 = _gn_affine(kp, p['kn_g'], p['kn_b'])
    y = _mm_pallas(_pad_rows(kp, Np_q), p['u2_W'], scale=sc2, shift=sh2)[:N_q]
    sc3, sh3 = _gn_affine(y, p['u2_g'], p['u2_b'])
    y = y * sc3 + sh3
    if strided:
        sc = jnp.max(s_feats[neighbors], axis=1)
    else:
        sc = s_feats
    if 'sc_W' in p:
        sc = _mm_pallas(_pad_rows(sc, Np_q), p['sc_W'])[:N_q]
        sc4, sh4 = _gn_affine(sc, p['sc_g'], p['sc_b'])
        sc = sc * sc4 + sh4
    return _lrelu_jnp(y + sc)


def kernel(points_0, points_1, points_2, neighbors_0, neighbors_1,
           neighbors_2, subsampling_0, subsampling_1, upsampling_0, params):
    N0 = points_0.shape[0]
    N1 = points_1.shape[0]
    N2 = points_2.shape[0]
    B = 512
    Np0, Np1, Np2 = _ceil_to(N0, B), _ceil_to(N1, B), _ceil_to(N2, B)

    p = params

    infl_e11 = _influence(points_0, points_0, neighbors_0,
                          p['e11']['kpts'], S0, Np0)
    infl_e12 = _influence(points_0, points_0, neighbors_0,
                          p['e12']['kpts'], S0, Np0)
    infl_l10 = _influence(points_1, points_0, subsampling_0,
                          p['l1_0']['kpts'], S0, Np1)
    infl_l11 = _influence(points_1, points_1, neighbors_1,
                          p['l1_1']['kpts'], 2 * S0, Np1)
    infl_l12 = _influence(points_1, points_1, neighbors_1,
                          p['l1_2']['kpts'], 2 * S0, Np1)
    infl_l20 = _influence(points_2, points_1, subsampling_1,
                          p['l2_0']['kpts'], 2 * S0, Np2)
    infl_l21 = _influence(points_2, points_2, neighbors_2,
                          p['l2_1']['kpts'], 4 * S0, Np2)
    infl_l22 = _influence(points_2, points_2, neighbors_2,
                          p['l2_2']['kpts'], 4 * S0, Np2)

    s_infl = infl_e11.sum(axis=0)
    w0 = jnp.pad(p['e11']['kp_W'][:, 0, :], ((0, XP - KS), (0, 0)))
    feats = _mm_pallas(s_infl, w0)[:N0]
    sc, sh = _gn_affine(feats, p['e11']['g'], p['e11']['b'])
    feats = _lrelu_jnp(feats * sc + sh)

    feats = _res_block(p['e12'], feats, infl_e12, neighbors_0, Np0, Np0,
                       strided=False)
    feats = _res_block(p['l1_0'], feats, infl_l10, subsampling_0, Np1, Np0,
                       strided=True)
    feats = _res_block(p['l1_1'], feats, infl_l11, neighbors_1, Np1, Np1,
                       strided=False)
    f1 = _res_block(p['l1_2'], feats, infl_l12, neighbors_1, Np1, Np1,
                    strided=False)
    feats = _res_block(p['l2_0'], f1, infl_l20, subsampling_1, Np2, Np1,
                       strided=True)
    feats = _res_block(p['l2_1'], feats, infl_l21, neighbors_2, Np2, Np2,
                       strided=False)
    f2 = _res_block(p['l2_2'], feats, infl_l22, neighbors_2, Np2, Np2,
                    strided=False)

    up = jnp.concatenate([f1, f2[upsampling_0[:, 0]]], axis=1)
    f1d = _mm_pallas(_pad_rows(up, Np1), p['dec0_W'])[:N1]
    scd, shd = _gn_affine(f1d, p['dec0_g'], p['dec0_b'])
    f1d = _lrelu_jnp(f1d * scd + shd)

    d2 = ((points_2[:, None, :] - points_1[None, :, :]) ** 2).sum(-1)
    _unused, idx = jax.lax.top_k(-d2, 32)
    gx = points_1[idx]
    gf = f1d[idx]
    rel = gx - points_2[:, None, :]
    h = _lrelu_jnp(jnp.concatenate([rel, gf], axis=-1) @ p['det_W1']
                   + p['det_b1'])
    scores = (h @ p['det_W2'] + p['det_b2'])[..., 0]
    attn = jax.nn.softmax(scores, axis=-1)
    xyz = jnp.einsum('mk,mki->mi', attn, gx)
    dist = jnp.sqrt(((gx - xyz[:, None, :]) ** 2).sum(-1) + 1e-12)
    sigma_out = jnp.einsum('mk,mk->m', attn, dist)[:, None]
    att_feat = jnp.einsum('mk,mkc->mc', attn, gf)
    g = jnp.max(_lrelu_jnp(gf @ p['desc_Wg']), axis=1)
    a = att_feat @ p['desc_Wa']
    desc = jnp.concatenate([g, a], axis=-1) @ p['desc_Wo']
    desc = desc / (jnp.linalg.norm(desc, axis=-1, keepdims=True) + 1e-8)
    return (f1d, f2, xyz, sigma_out, desc)
